# Initial kernel scaffold; baseline (speedup 1.0000x reference)
import jax
import jax.numpy as jnp
from jax.experimental import pallas as pl


def kernel(x, W1, g1, b1, W2, g2, b2, W3, g3, b3, W4, g4, b4, W5, g5, b5, W6, g6, b6):
    raise NotImplementedError("write your pallas kernel here")


# speedup vs baseline: 6.5347x; 6.5347x over previous
import functools

import jax
import jax.numpy as jnp
from jax import lax
from jax.experimental import pallas as pl
from jax.experimental.pallas import tpu as pltpu
from jax.experimental.pallas import tpu_sc as plsc

B = 8
N = 2048
KNN = 20
BN = B * N
M1 = BN * KNN
NEG = -3.0e38
GW = 128


def _mm(a, b):
    return lax.dot_general(a, b, (((1,), (0,)), ((), ())),
                           preferred_element_type=jnp.float32)


def _lrelu(x):
    return jnp.where(x >= 0, x, 0.2 * x)


def _knn_kernel(rows_ref, full_ref, nsq_ref, out_ref, *, tm, n, k):
    rows = rows_ref[0].astype(jnp.bfloat16)
    full = full_ref[0].astype(jnp.bfloat16)
    raw = lax.dot_general(
        rows, full, (((1,), (1,)), ((), ())),
        preferred_element_type=jnp.float32)
    inner = -2.0 * raw
    pd = nsq_ref[0, 0:1, :] - inner
    cols = lax.broadcasted_iota(jnp.int32, (tm, n), 1)
    base = pl.program_id(0) * n
    outs = []
    for _ in range(k):
        mx = jnp.max(pd, axis=1, keepdims=True)
        idx = jnp.min(jnp.where(pd == mx, cols, n), axis=1, keepdims=True)
        pd = jnp.where(cols == idx, NEG, pd)
        outs.append(idx)
    out_ref[0] = jnp.concatenate(outs, axis=1) + base


def _knn(xtp):
    c = xtp.shape[-1]
    tm = 256
    nsq = -jnp.sum(xtp * xtp, axis=2)
    nsqb = jnp.broadcast_to(nsq[:, None, :], (B, 8, N))
    return pl.pallas_call(
        functools.partial(_knn_kernel, tm=tm, n=N, k=KNN),
        grid=(B, N // tm),
        in_specs=[
            pl.BlockSpec((1, tm, c), lambda b, i: (b, i, 0)),
            pl.BlockSpec((1, N, c), lambda b, i: (b, 0, 0)),
            pl.BlockSpec((1, 8, N), lambda b, i: (b, 0, 0)),
        ],
        out_specs=pl.BlockSpec((1, tm, KNN), lambda b, i: (b, i, 0)),
        out_shape=jax.ShapeDtypeStruct((B, N, KNN), jnp.int32),
    )(xtp, xtp, nsqb)


def _sc_gather(table, idx):
    m, d = idx.shape[0], table.shape[1]
    info = plsc.get_sparse_core_info()
    nc, ns = info.num_cores, info.num_subcores
    nw = nc * ns
    bpw = m // nw
    ch = 512
    nchunks = bpw // ch
    mesh = plsc.VectorSubcoreMesh(core_axis_name="c", subcore_axis_name="s")

    @functools.partial(
        pl.kernel, mesh=mesh,
        out_type=jax.ShapeDtypeStruct((m, d), jnp.float32),
        scratch_types=[
            pltpu.VMEM((ch,), jnp.int32),
            pltpu.VMEM((ch, d), jnp.float32),
            pltpu.SemaphoreType.DMA,
        ])
    def k(table_hbm, idx_hbm, out_hbm, idx_v, rows_v, sem):
        wid = lax.axis_index("s") * nc + lax.axis_index("c")
        for j in range(nchunks):
            base = wid * bpw + j * ch
            pltpu.sync_copy(idx_hbm.at[pl.ds(base, ch)], idx_v)
            pltpu.async_copy(table_hbm.at[idx_v], rows_v, sem).wait()
            pltpu.sync_copy(rows_v, out_hbm.at[pl.ds(base, ch)])

    return k(table, idx)


def _edge_a(feat_ref, xt_ref, j, cpad):
    xc = xt_ref[...]
    d = feat_ref[:, j, 0:cpad] - xc
    return jnp.concatenate(
        [d.astype(jnp.bfloat16), xc.astype(jnp.bfloat16)], axis=1)


def _stats1_kernel(feat_ref, xt_ref, w1_ref, out_ref, *, k, cpad):
    @pl.when(pl.program_id(0) == 0)
    def _():
        out_ref[...] = jnp.zeros_like(out_ref)
    o = w1_ref.shape[1]
    s = jnp.zeros((1, o), jnp.float32)
    q = jnp.zeros((1, o), jnp.float32)
    for j in range(k):
        z = _mm(_edge_a(feat_ref, xt_ref, j, cpad), w1_ref[...])
        s = s + jnp.sum(z, axis=0, keepdims=True)
        q = q + jnp.sum(z * z, axis=0, keepdims=True)
    out_ref[0:1, :] += s
    out_ref[1:2, :] += q


def _stats2_kernel(feat_ref, xt_ref, w1_ref, sb1_ref, w2_ref, out_ref, *,
                   k, cpad):
    @pl.when(pl.program_id(0) == 0)
    def _():
        out_ref[...] = jnp.zeros_like(out_ref)
    o = w2_ref.shape[1]
    s = jnp.zeros((1, o), jnp.float32)
    q = jnp.zeros((1, o), jnp.float32)
    for j in range(k):
        z1 = _mm(_edge_a(feat_ref, xt_ref, j, cpad), w1_ref[...])
        a1 = _lrelu(z1 * sb1_ref[0:1, :] + sb1_ref[1:2, :])
        z2 = _mm(a1.astype(jnp.bfloat16), w2_ref[...])
        s = s + jnp.sum(z2, axis=0, keepdims=True)
        q = q + jnp.sum(z2 * z2, axis=0, keepdims=True)
    out_ref[0:1, :] += s
    out_ref[1:2, :] += q


def _apply2_kernel(feat_ref, xt_ref, w1_ref, sb1_ref, w2_ref, sb2_ref,
                   out_ref, *, k, cpad):
    acc = None
    for j in range(k):
        z1 = _mm(_edge_a(feat_ref, xt_ref, j, cpad), w1_ref[...])
        a1 = _lrelu(z1 * sb1_ref[0:1, :] + sb1_ref[1:2, :])
        z2 = _mm(a1.astype(jnp.bfloat16), w2_ref[...])
        a2 = _lrelu(z2 * sb2_ref[0:1, :] + sb2_ref[1:2, :])
        acc = a2 if acc is None else jnp.maximum(acc, a2)
    out_ref[...] = acc


def _apply1_kernel(feat_ref, xt_ref, w1_ref, sb1_ref, out_ref, *, k, cpad):
    acc = None
    for j in range(k):
        z1 = _mm(_edge_a(feat_ref, xt_ref, j, cpad), w1_ref[...])
        a1 = _lrelu(z1 * sb1_ref[0:1, :] + sb1_ref[1:2, :])
        acc = a1 if acc is None else jnp.maximum(acc, a1)
    out_ref[...] = acc


_TN = 256


def _wspec(shape):
    return pl.BlockSpec(shape, lambda i: tuple(0 for _ in shape))


def _edge_specs(c, extra):
    return [
        pl.BlockSpec((_TN, KNN, GW), lambda i: (i, 0, 0)),
        pl.BlockSpec((_TN, c), lambda i: (i, 0)),
    ] + extra


def _stats1(feat, flat, w1):
    c = flat.shape[1]
    o = w1.shape[1]
    return pl.pallas_call(
        functools.partial(_stats1_kernel, k=KNN, cpad=c),
        grid=(BN // _TN,),
        in_specs=_edge_specs(c, [_wspec(w1.shape)]),
        out_specs=_wspec((8, o)),
        out_shape=jax.ShapeDtypeStruct((8, o), jnp.float32),
    )(feat, flat, w1)


def _stats2(feat, flat, w1, sb1, w2):
    c = flat.shape[1]
    o2 = w2.shape[1]
    return pl.pallas_call(
        functools.partial(_stats2_kernel, k=KNN, cpad=c),
        grid=(BN // _TN,),
        in_specs=_edge_specs(c, [_wspec(w1.shape), _wspec(sb1.shape),
                                 _wspec(w2.shape)]),
        out_specs=_wspec((8, o2)),
        out_shape=jax.ShapeDtypeStruct((8, o2), jnp.float32),
    )(feat, flat, w1, sb1, w2)


def _apply2(feat, flat, w1, sb1, w2, sb2):
    c = flat.shape[1]
    o2 = w2.shape[1]
    return pl.pallas_call(
        functools.partial(_apply2_kernel, k=KNN, cpad=c),
        grid=(BN // _TN,),
        in_specs=_edge_specs(c, [_wspec(w1.shape), _wspec(sb1.shape),
                                 _wspec(w2.shape), _wspec(sb2.shape)]),
        out_specs=pl.BlockSpec((_TN, o2), lambda i: (i, 0)),
        out_shape=jax.ShapeDtypeStruct((BN, o2), jnp.float32),
    )(feat, flat, w1, sb1, w2, sb2)


def _apply1(feat, flat, w1, sb1):
    c = flat.shape[1]
    o = w1.shape[1]
    return pl.pallas_call(
        functools.partial(_apply1_kernel, k=KNN, cpad=c),
        grid=(BN // _TN,),
        in_specs=_edge_specs(c, [_wspec(w1.shape), _wspec(sb1.shape)]),
        out_specs=pl.BlockSpec((_TN, o), lambda i: (i, 0)),
        out_shape=jax.ShapeDtypeStruct((BN, o), jnp.float32),
    )(feat, flat, w1, sb1)


def _head_stats_kernel(h_ref, w_ref, out_ref):
    @pl.when(pl.program_id(0) == 0)
    def _():
        out_ref[...] = jnp.zeros_like(out_ref)
    z = _mm(h_ref[...].astype(jnp.bfloat16), w_ref[...])
    out_ref[0:1, :] += jnp.sum(z, axis=0, keepdims=True)
    out_ref[1:2, :] += jnp.sum(z * z, axis=0, keepdims=True)


def _head_apply_kernel(h_ref, w_ref, sb_ref, out_ref):
    z = _mm(h_ref[...].astype(jnp.bfloat16), w_ref[...])
    out_ref[...] = _lrelu(z * sb_ref[0:1, :] + sb_ref[1:2, :])


def _head_stats(h, wt):
    c, o = wt.shape
    return pl.pallas_call(
        _head_stats_kernel,
        grid=(BN // _TN,),
        in_specs=[pl.BlockSpec((_TN, c), lambda i: (i, 0)), _wspec((c, o))],
        out_specs=_wspec((8, o)),
        out_shape=jax.ShapeDtypeStruct((8, o), jnp.float32),
    )(h, wt)


def _head_apply(h, wt, sb):
    c, o = wt.shape
    return pl.pallas_call(
        _head_apply_kernel,
        grid=(BN // _TN,),
        in_specs=[pl.BlockSpec((_TN, c), lambda i: (i, 0)), _wspec((c, o)),
                  _wspec((8, o))],
        out_specs=pl.BlockSpec((_TN, o), lambda i: (i, 0)),
        out_shape=jax.ShapeDtypeStruct((BN, o), jnp.float32),
    )(h, wt, sb)


def _fold(sq, m_count, g, b):
    mean = sq[0, :] / m_count
    var = sq[1, :] / m_count - mean * mean
    scale = g * lax.rsqrt(var + 1e-5)
    bias = b - mean * scale
    return jnp.pad(jnp.stack([scale, bias], axis=0), ((0, 6), (0, 0)))


def _edge_w(w, cin):
    cpad = max(cin, 16)
    wa = jnp.pad(w[:, :cin].T, ((0, cpad - cin), (0, 0)))
    wb = jnp.pad(w[:, cin:].T, ((0, cpad - cin), (0, 0)))
    return jnp.concatenate([wa, wb], axis=0).astype(jnp.bfloat16)


def _edge_block2(flat, xtp, w_first, g1_, b1_, w_second, g2_, b2_, cin):
    cpad = flat.shape[1]
    tab = jnp.pad(flat, ((0, 0), (0, GW - cpad)))
    idx = _knn(xtp).reshape(M1)
    feat = _sc_gather(tab, idx).reshape(BN, KNN, GW)
    w1 = _edge_w(w_first, cin)
    sq1 = _stats1(feat, flat, w1)
    sb1 = _fold(sq1, M1, g1_, b1_)
    w2 = w_second.T.astype(jnp.bfloat16)
    sq2 = _stats2(feat, flat, w1, sb1, w2)
    sb2 = _fold(sq2, M1, g2_, b2_)
    return _apply2(feat, flat, w1, sb1, w2, sb2)


def _edge_block1(flat, xtp, w_first, g1_, b1_, cin):
    cpad = flat.shape[1]
    tab = jnp.pad(flat, ((0, 0), (0, GW - cpad)))
    idx = _knn(xtp).reshape(M1)
    feat = _sc_gather(tab, idx).reshape(BN, KNN, GW)
    w1 = _edge_w(w_first, cin)
    sq1 = _stats1(feat, flat, w1)
    sb1 = _fold(sq1, M1, g1_, b1_)
    return _apply1(feat, flat, w1, sb1)


def kernel(x, W1, g1, b1, W2, g2, b2, W3, g3, b3, W4, g4, b4,
           W5, g5, b5, W6, g6, b6):
    xt1 = jnp.transpose(x, (0, 2, 1))
    xt1p = jnp.pad(xt1, ((0, 0), (0, 0), (0, 13)))
    flat1 = xt1p.reshape(BN, 16)
    x1 = _edge_block2(flat1, xt1p, W1, g1, b1, W2, g2, b2, cin=3)
    x2 = _edge_block2(x1, x1.reshape(B, N, 64), W3, g3, b3, W4, g4, b4,
                      cin=64)
    x3 = _edge_block1(x2, x2.reshape(B, N, 64), W5, g5, b5, cin=64)
    h = jnp.concatenate([x1, x2, x3], axis=1)
    w6 = W6.T.astype(jnp.bfloat16)
    sq6 = _head_stats(h, w6)
    sb6 = _fold(sq6, BN, g6, b6)
    out = _head_apply(h, w6, sb6)
    return out.reshape(B, N, 512)


# speedup vs baseline: 6.7924x; 1.0394x over previous
import functools

import jax
import jax.numpy as jnp
from jax import lax
from jax.experimental import pallas as pl
from jax.experimental.pallas import tpu as pltpu
from jax.experimental.pallas import tpu_sc as plsc

B = 8
N = 2048
KNN = 20
BN = B * N
M1 = BN * KNN
NEG = -3.0e38
GW = 128


def _mm(a, b):
    return lax.dot_general(a, b, (((1,), (0,)), ((), ())),
                           preferred_element_type=jnp.float32)


def _lrelu(x):
    return jnp.where(x >= 0, x, 0.2 * x)


def _knn_kernel(rows_ref, full_ref, nsq_ref, out_ref, *, tm, n, k):
    rows = rows_ref[0].astype(jnp.bfloat16)
    full = full_ref[0].astype(jnp.bfloat16)
    raw = lax.dot_general(
        rows, full, (((1,), (1,)), ((), ())),
        preferred_element_type=jnp.float32)
    inner = -2.0 * raw
    pd = nsq_ref[0, 0:1, :] - inner
    cols = lax.broadcasted_iota(jnp.int32, (tm, n), 1)
    base = pl.program_id(0) * n
    outs = []
    for _ in range(k):
        mx = jnp.max(pd, axis=1, keepdims=True)
        idx = jnp.min(jnp.where(pd == mx, cols, n), axis=1, keepdims=True)
        pd = jnp.where(cols == idx, NEG, pd)
        outs.append(idx)
    out_ref[0] = jnp.concatenate(outs, axis=1) + base


def _knn(xtp):
    c = xtp.shape[-1]
    tm = 512
    nsq = -jnp.sum(xtp * xtp, axis=2)
    nsqb = jnp.broadcast_to(nsq[:, None, :], (B, 8, N))
    return pl.pallas_call(
        functools.partial(_knn_kernel, tm=tm, n=N, k=KNN),
        grid=(B, N // tm),
        in_specs=[
            pl.BlockSpec((1, tm, c), lambda b, i: (b, i, 0)),
            pl.BlockSpec((1, N, c), lambda b, i: (b, 0, 0)),
            pl.BlockSpec((1, 8, N), lambda b, i: (b, 0, 0)),
        ],
        out_specs=pl.BlockSpec((1, tm, KNN), lambda b, i: (b, i, 0)),
        out_shape=jax.ShapeDtypeStruct((B, N, KNN), jnp.int32),
    )(xtp, xtp, nsqb)


def _sc_gather(table, idx):
    m, d = idx.shape[0], table.shape[1]
    info = plsc.get_sparse_core_info()
    nc, ns = info.num_cores, info.num_subcores
    nw = nc * ns
    bpw = m // nw
    ch = 512
    nchunks = bpw // ch
    mesh = plsc.VectorSubcoreMesh(core_axis_name="c", subcore_axis_name="s")

    @functools.partial(
        pl.kernel, mesh=mesh,
        out_type=jax.ShapeDtypeStruct((m, d), jnp.float32),
        scratch_types=[
            pltpu.VMEM((ch,), jnp.int32),
            pltpu.VMEM((ch, d), jnp.float32),
            pltpu.SemaphoreType.DMA,
        ])
    def k(table_hbm, idx_hbm, out_hbm, idx_v, rows_v, sem):
        wid = lax.axis_index("s") * nc + lax.axis_index("c")
        for j in range(nchunks):
            base = wid * bpw + j * ch
            pltpu.sync_copy(idx_hbm.at[pl.ds(base, ch)], idx_v)
            pltpu.async_copy(table_hbm.at[idx_v], rows_v, sem).wait()
            pltpu.sync_copy(rows_v, out_hbm.at[pl.ds(base, ch)])

    return k(table, idx)


def _edge_a_all(feat_ref, xt_ref, k, cpad):
    xc = xt_ref[...]
    xcb = xc.astype(jnp.bfloat16)
    parts = []
    for j in range(k):
        d = feat_ref[:, j, 0:cpad] - xc
        parts.append(jnp.concatenate([d.astype(jnp.bfloat16), xcb], axis=1))
    return jnp.concatenate(parts, axis=0)


def _kmax(z, k, tn):
    acc = z[0:tn, :]
    for j in range(1, k):
        acc = jnp.maximum(acc, z[j * tn:(j + 1) * tn, :])
    return acc


def _stats1_kernel(feat_ref, xt_ref, w1_ref, out_ref, *, k, cpad):
    @pl.when(pl.program_id(0) == 0)
    def _():
        out_ref[...] = jnp.zeros_like(out_ref)
    z = _mm(_edge_a_all(feat_ref, xt_ref, k, cpad), w1_ref[...])
    out_ref[0:1, :] += jnp.sum(z, axis=0, keepdims=True)
    out_ref[1:2, :] += jnp.sum(z * z, axis=0, keepdims=True)


def _stats2_kernel(feat_ref, xt_ref, w1_ref, sb1_ref, w2_ref, out_ref, *,
                   k, cpad):
    @pl.when(pl.program_id(0) == 0)
    def _():
        out_ref[...] = jnp.zeros_like(out_ref)
    z1 = _mm(_edge_a_all(feat_ref, xt_ref, k, cpad), w1_ref[...])
    a1 = _lrelu(z1 * sb1_ref[0:1, :] + sb1_ref[1:2, :])
    z2 = _mm(a1.astype(jnp.bfloat16), w2_ref[...])
    out_ref[0:1, :] += jnp.sum(z2, axis=0, keepdims=True)
    out_ref[1:2, :] += jnp.sum(z2 * z2, axis=0, keepdims=True)


def _apply2_kernel(feat_ref, xt_ref, w1_ref, sb1_ref, w2_ref, sb2_ref,
                   out_ref, *, k, cpad):
    tn = xt_ref.shape[0]
    z1 = _mm(_edge_a_all(feat_ref, xt_ref, k, cpad), w1_ref[...])
    a1 = _lrelu(z1 * sb1_ref[0:1, :] + sb1_ref[1:2, :])
    z2 = _mm(a1.astype(jnp.bfloat16), w2_ref[...])
    a2 = _lrelu(z2 * sb2_ref[0:1, :] + sb2_ref[1:2, :])
    out_ref[...] = _kmax(a2, k, tn)


def _apply1_kernel(feat_ref, xt_ref, w1_ref, sb1_ref, out_ref, *, k, cpad):
    tn = xt_ref.shape[0]
    z1 = _mm(_edge_a_all(feat_ref, xt_ref, k, cpad), w1_ref[...])
    a1 = _lrelu(z1 * sb1_ref[0:1, :] + sb1_ref[1:2, :])
    out_ref[...] = _kmax(a1, k, tn)


_TN = 256


def _wspec(shape):
    return pl.BlockSpec(shape, lambda i: tuple(0 for _ in shape))


def _edge_specs(c, extra):
    return [
        pl.BlockSpec((_TN, KNN, GW), lambda i: (i, 0, 0)),
        pl.BlockSpec((_TN, c), lambda i: (i, 0)),
    ] + extra


def _stats1(feat, flat, w1):
    c = flat.shape[1]
    o = w1.shape[1]
    return pl.pallas_call(
        functools.partial(_stats1_kernel, k=KNN, cpad=c),
        grid=(BN // _TN,),
        in_specs=_edge_specs(c, [_wspec(w1.shape)]),
        out_specs=_wspec((8, o)),
        out_shape=jax.ShapeDtypeStruct((8, o), jnp.float32),
    )(feat, flat, w1)


def _stats2(feat, flat, w1, sb1, w2):
    c = flat.shape[1]
    o2 = w2.shape[1]
    return pl.pallas_call(
        functools.partial(_stats2_kernel, k=KNN, cpad=c),
        grid=(BN // _TN,),
        in_specs=_edge_specs(c, [_wspec(w1.shape), _wspec(sb1.shape),
                                 _wspec(w2.shape)]),
        out_specs=_wspec((8, o2)),
        out_shape=jax.ShapeDtypeStruct((8, o2), jnp.float32),
    )(feat, flat, w1, sb1, w2)


def _apply2(feat, flat, w1, sb1, w2, sb2):
    c = flat.shape[1]
    o2 = w2.shape[1]
    return pl.pallas_call(
        functools.partial(_apply2_kernel, k=KNN, cpad=c),
        grid=(BN // _TN,),
        in_specs=_edge_specs(c, [_wspec(w1.shape), _wspec(sb1.shape),
                                 _wspec(w2.shape), _wspec(sb2.shape)]),
        out_specs=pl.BlockSpec((_TN, o2), lambda i: (i, 0)),
        out_shape=jax.ShapeDtypeStruct((BN, o2), jnp.float32),
    )(feat, flat, w1, sb1, w2, sb2)


def _apply1(feat, flat, w1, sb1):
    c = flat.shape[1]
    o = w1.shape[1]
    return pl.pallas_call(
        functools.partial(_apply1_kernel, k=KNN, cpad=c),
        grid=(BN // _TN,),
        in_specs=_edge_specs(c, [_wspec(w1.shape), _wspec(sb1.shape)]),
        out_specs=pl.BlockSpec((_TN, o), lambda i: (i, 0)),
        out_shape=jax.ShapeDtypeStruct((BN, o), jnp.float32),
    )(feat, flat, w1, sb1)


def _head_stats_kernel(h_ref, w_ref, out_ref):
    @pl.when(pl.program_id(0) == 0)
    def _():
        out_ref[...] = jnp.zeros_like(out_ref)
    z = _mm(h_ref[...].astype(jnp.bfloat16), w_ref[...])
    out_ref[0:1, :] += jnp.sum(z, axis=0, keepdims=True)
    out_ref[1:2, :] += jnp.sum(z * z, axis=0, keepdims=True)


def _head_apply_kernel(h_ref, w_ref, sb_ref, out_ref):
    z = _mm(h_ref[...].astype(jnp.bfloat16), w_ref[...])
    out_ref[...] = _lrelu(z * sb_ref[0:1, :] + sb_ref[1:2, :])


def _head_stats(h, wt):
    c, o = wt.shape
    return pl.pallas_call(
        _head_stats_kernel,
        grid=(BN // _TN,),
        in_specs=[pl.BlockSpec((_TN, c), lambda i: (i, 0)), _wspec((c, o))],
        out_specs=_wspec((8, o)),
        out_shape=jax.ShapeDtypeStruct((8, o), jnp.float32),
    )(h, wt)


def _head_apply(h, wt, sb):
    c, o = wt.shape
    return pl.pallas_call(
        _head_apply_kernel,
        grid=(BN // _TN,),
        in_specs=[pl.BlockSpec((_TN, c), lambda i: (i, 0)), _wspec((c, o)),
                  _wspec((8, o))],
        out_specs=pl.BlockSpec((_TN, o), lambda i: (i, 0)),
        out_shape=jax.ShapeDtypeStruct((BN, o), jnp.float32),
    )(h, wt, sb)


def _fold(sq, m_count, g, b):
    mean = sq[0, :] / m_count
    var = sq[1, :] / m_count - mean * mean
    scale = g * lax.rsqrt(var + 1e-5)
    bias = b - mean * scale
    return jnp.pad(jnp.stack([scale, bias], axis=0), ((0, 6), (0, 0)))


def _edge_w(w, cin):
    cpad = max(cin, 16)
    wa = jnp.pad(w[:, :cin].T, ((0, cpad - cin), (0, 0)))
    wb = jnp.pad(w[:, cin:].T, ((0, cpad - cin), (0, 0)))
    return jnp.concatenate([wa, wb], axis=0).astype(jnp.bfloat16)


def _edge_block2(flat, xtp, w_first, g1_, b1_, w_second, g2_, b2_, cin):
    cpad = flat.shape[1]
    tab = jnp.pad(flat, ((0, 0), (0, GW - cpad)))
    idx = _knn(xtp).reshape(M1)
    feat = _sc_gather(tab, idx).reshape(BN, KNN, GW)
    w1 = _edge_w(w_first, cin)
    sq1 = _stats1(feat, flat, w1)
    sb1 = _fold(sq1, M1, g1_, b1_)
    w2 = w_second.T.astype(jnp.bfloat16)
    sq2 = _stats2(feat, flat, w1, sb1, w2)
    sb2 = _fold(sq2, M1, g2_, b2_)
    return _apply2(feat, flat, w1, sb1, w2, sb2)


def _edge_block1(flat, xtp, w_first, g1_, b1_, cin):
    cpad = flat.shape[1]
    tab = jnp.pad(flat, ((0, 0), (0, GW - cpad)))
    idx = _knn(xtp).reshape(M1)
    feat = _sc_gather(tab, idx).reshape(BN, KNN, GW)
    w1 = _edge_w(w_first, cin)
    sq1 = _stats1(feat, flat, w1)
    sb1 = _fold(sq1, M1, g1_, b1_)
    return _apply1(feat, flat, w1, sb1)


def kernel(x, W1, g1, b1, W2, g2, b2, W3, g3, b3, W4, g4, b4,
           W5, g5, b5, W6, g6, b6):
    xt1 = jnp.transpose(x, (0, 2, 1))
    xt1p = jnp.pad(xt1, ((0, 0), (0, 0), (0, 13)))
    flat1 = xt1p.reshape(BN, 16)
    x1 = _edge_block2(flat1, xt1p, W1, g1, b1, W2, g2, b2, cin=3)
    x2 = _edge_block2(x1, x1.reshape(B, N, 64), W3, g3, b3, W4, g4, b4,
                      cin=64)
    x3 = _edge_block1(x2, x2.reshape(B, N, 64), W5, g5, b5, cin=64)
    h = jnp.concatenate([x1, x2, x3], axis=1)
    w6 = W6.T.astype(jnp.bfloat16)
    sq6 = _head_stats(h, w6)
    sb6 = _fold(sq6, BN, g6, b6)
    out = _head_apply(h, w6, sb6)
    return out.reshape(B, N, 512)
